# baseline (speedup 1.0000x reference)
import jax
import jax.numpy as jnp
from jax import lax
from jax.experimental import pallas as pl
from jax.experimental.pallas import tpu as pltpu
from jax.experimental.pallas import tpu_sc as plsc

F = 16
B = 16384
NW = 32
BPW = B // NW
CHUNK = 128
NCHUNK = BPW // CHUNK
GPC = CHUNK // F
RPB = 128 // F


def _body(users, pos_items, neg_items, ue, ie, ub, ib, out,
          idx_u, idx_p, idx_n, blk_u, blk_p, blk_n,
          bufs_u, bufs_p, bufs_n, bu, bp, bn, out_v,
          sem0, sem1, bsem):
  wid = lax.axis_index("c") * 16 + lax.axis_index("s")
  base = wid * BPW

  pltpu.sync_copy(users.at[pl.ds(base, BPW)], idx_u)
  pltpu.sync_copy(pos_items.at[pl.ds(base, BPW)], idx_p)
  pltpu.sync_copy(neg_items.at[pl.ds(base, BPW)], idx_n)

  for idx, blk in ((idx_u, blk_u), (idx_p, blk_p), (idx_n, blk_n)):
    for i in range(BPW // F):
      sl = pl.ds(i * F, F)
      blk[sl] = lax.shift_right_logical(idx[sl], 3)

  bias_copies = []
  for idx, table, dst in ((idx_u, ub, bu), (idx_p, ib, bp), (idx_n, ib, bn)):
    for j in range(NCHUNK):
      sl = pl.ds(j * CHUNK, CHUNK)
      bias_copies.append(
          pltpu.async_copy(table.at[idx.at[sl]], dst.at[sl], bsem))

  sems = (sem0, sem1)

  def fire(c):
    slot = c % 2
    sl = pl.ds(c * CHUNK, CHUNK)
    return [
        pltpu.async_copy(ue.at[blk_u.at[sl]], bufs_u.at[slot], sems[slot]),
        pltpu.async_copy(ie.at[blk_p.at[sl]], bufs_p.at[slot], sems[slot]),
        pltpu.async_copy(ie.at[blk_n.at[sl]], bufs_n.at[slot], sems[slot]),
    ]

  lanes = lax.iota(jnp.int32, F)

  def compute(c):
    slot = c % 2
    for g in range(GPC):
      o = c * CHUNK + g * F
      row = g * F + lanes
      vu = idx_u[pl.ds(o, F)]
      vp = idx_p[pl.ds(o, F)]
      vn = idx_n[pl.ds(o, F)]
      cu = (vu & 7) * F
      cp = (vp & 7) * F
      cn = (vn & 7) * F
      acc = bu[pl.ds(o, F)] + bp[pl.ds(o, F)] - bn[pl.ds(o, F)]
      for f in range(F):
        u = plsc.load_gather(bufs_u.at[slot], [row, cu + f])
        p = plsc.load_gather(bufs_p.at[slot], [row, cp + f])
        n = plsc.load_gather(bufs_n.at[slot], [row, cn + f])
        acc = acc + u * (p - n)
      out_v[pl.ds(o, F)] = acc

  inflight = fire(0)
  for c in bias_copies:
    c.wait()
  for c in range(NCHUNK):
    nxt = fire(c + 1) if c + 1 < NCHUNK else []
    for d in inflight:
      d.wait()
    compute(c)
    inflight = nxt

  pltpu.sync_copy(out_v, out.at[pl.ds(base, BPW)])


@jax.jit
def kernel(users, pos_items, neg_items, user_embeddings, item_embeddings,
           user_biases, item_biases):
  mesh = plsc.VectorSubcoreMesh(core_axis_name="c", subcore_axis_name="s")
  run = pl.kernel(
      _body,
      out_type=jax.ShapeDtypeStruct((B,), jnp.float32),
      mesh=mesh,
      scratch_types=[
          pltpu.VMEM((BPW,), jnp.int32),
          pltpu.VMEM((BPW,), jnp.int32),
          pltpu.VMEM((BPW,), jnp.int32),
          pltpu.VMEM((BPW,), jnp.int32),
          pltpu.VMEM((BPW,), jnp.int32),
          pltpu.VMEM((BPW,), jnp.int32),
          pltpu.VMEM((2, CHUNK, 128), jnp.float32),
          pltpu.VMEM((2, CHUNK, 128), jnp.float32),
          pltpu.VMEM((2, CHUNK, 128), jnp.float32),
          pltpu.VMEM((BPW,), jnp.float32),
          pltpu.VMEM((BPW,), jnp.float32),
          pltpu.VMEM((BPW,), jnp.float32),
          pltpu.VMEM((BPW,), jnp.float32),
          pltpu.SemaphoreType.DMA,
          pltpu.SemaphoreType.DMA,
          pltpu.SemaphoreType.DMA,
      ],
      compiler_params=pltpu.CompilerParams(needs_layout_passes=False,
                                           use_tc_tiling_on_sc=True),
  )
  def blocked(t):
    return jnp.concatenate([t[k::8] for k in range(8)], axis=1)

  return run(users.astype(jnp.int32), pos_items.astype(jnp.int32),
             neg_items.astype(jnp.int32),
             blocked(user_embeddings), blocked(item_embeddings),
             user_biases.reshape(-1), item_biases.reshape(-1))


# speedup vs baseline: 6.2173x; 6.2173x over previous
import jax
import jax.numpy as jnp
from jax import lax
from jax.experimental import pallas as pl
from jax.experimental.pallas import tpu as pltpu
from jax.experimental.pallas import tpu_sc as plsc

F = 16
B = 16384
NW = 32
BPW = B // NW
CHUNK = 128
NCHUNK = BPW // CHUNK
GROUPS = BPW // F


def _body(users, pos_items, neg_items, ue, ie, ub, ib, out,
          idx_u, idx_p, idx_n, rows_u, rows_p, rows_n, bu, bp, bn, out_v,
          sem):
  wid = lax.axis_index("c") * 16 + lax.axis_index("s")
  base = wid * BPW

  pltpu.sync_copy(users.at[pl.ds(base, BPW)], idx_u)
  pltpu.sync_copy(pos_items.at[pl.ds(base, BPW)], idx_p)
  pltpu.sync_copy(neg_items.at[pl.ds(base, BPW)], idx_n)

  copies = []
  for idx, table, dst in ((idx_u, ue, rows_u), (idx_p, ie, rows_p),
                          (idx_n, ie, rows_n), (idx_u, ub, bu),
                          (idx_p, ib, bp), (idx_n, ib, bn)):
    for j in range(NCHUNK):
      sl = pl.ds(j * CHUNK, CHUNK)
      copies.append(pltpu.async_copy(table.at[idx.at[sl]], dst.at[sl], sem))
  for c in copies:
    c.wait()

  lanes = lax.iota(jnp.int32, F)

  def group(g, carry):
    bidx = g * F + lanes
    acc = bu[pl.ds(g * F, F)] + bp[pl.ds(g * F, F)] - bn[pl.ds(g * F, F)]
    for f in range(F):
      fvec = jnp.full((F,), f, jnp.int32)
      u = plsc.load_gather(rows_u, [bidx, fvec])
      p = plsc.load_gather(rows_p, [bidx, fvec])
      n = plsc.load_gather(rows_n, [bidx, fvec])
      acc = acc + u * (p - n)
    out_v[pl.ds(g * F, F)] = acc
    return carry

  lax.fori_loop(0, GROUPS, group, 0)
  pltpu.sync_copy(out_v, out.at[pl.ds(base, BPW)])


@jax.jit
def kernel(users, pos_items, neg_items, user_embeddings, item_embeddings,
           user_biases, item_biases):
  mesh = plsc.VectorSubcoreMesh(core_axis_name="c", subcore_axis_name="s")
  run = pl.kernel(
      _body,
      out_type=jax.ShapeDtypeStruct((B,), jnp.float32),
      mesh=mesh,
      scratch_types=[
          pltpu.VMEM((BPW,), jnp.int32),
          pltpu.VMEM((BPW,), jnp.int32),
          pltpu.VMEM((BPW,), jnp.int32),
          pltpu.VMEM((BPW, F), jnp.float32),
          pltpu.VMEM((BPW, F), jnp.float32),
          pltpu.VMEM((BPW, F), jnp.float32),
          pltpu.VMEM((BPW,), jnp.float32),
          pltpu.VMEM((BPW,), jnp.float32),
          pltpu.VMEM((BPW,), jnp.float32),
          pltpu.VMEM((BPW,), jnp.float32),
          pltpu.SemaphoreType.DMA,
      ],
      compiler_params=pltpu.CompilerParams(needs_layout_passes=False,
                                           use_tc_tiling_on_sc=False),
  )
  return run(users.astype(jnp.int32), pos_items.astype(jnp.int32),
             neg_items.astype(jnp.int32), user_embeddings, item_embeddings,
             user_biases.reshape(-1), item_biases.reshape(-1))
